# Initial kernel scaffold; baseline (speedup 1.0000x reference)
import jax
import jax.numpy as jnp
from jax.experimental import pallas as pl


def kernel(inputs, adj_mx, hidden_state, Wg0, bg0, Wc0, bc0, Wg1, bg1, Wc1, bc1, Wp, bp):
    raise NotImplementedError("write your pallas kernel here")


# speedup vs baseline: 3.8872x; 3.8872x over previous
import jax
import jax.numpy as jnp
from jax.experimental import pallas as pl

N = 512
U = 64
B = 32
BC = 8
NTAP = 3
F32 = jnp.float32


def _split_w(W, xdim):
    F = xdim + U
    out = W.shape[1]
    W3 = W.reshape(F, NTAP, out)
    Wx = jnp.transpose(W3[:xdim], (1, 0, 2))
    Wh = jnp.transpose(W3[xdim:], (1, 0, 2))
    return Wx, Wh


def _kron_x0(Wx):
    eye = jnp.eye(BC, dtype=F32)
    return jax.vmap(lambda w: jnp.kron(eye, w))(Wx)


def _dcgru_body(inp_ref, adj_ref, hid_ref,
                kxg0_ref, whg0_ref, kxc0_ref, whc0_ref, bg0_ref, bc0_ref,
                wxg1_ref, whg1_ref, wxc1_ref, whc1_ref, bg1_ref, bc1_ref,
                wp_ref, bp_ref,
                out_ref, hs_ref):
    A = adj_ref[...]

    def mm(a, b):
        return jax.lax.dot_general(a, b, (((1,), (0,)), ((), ())),
                                   preferred_element_type=F32)

    def diffuse(z_nm):
        z1 = mm(A, z_nm)
        z2 = 2.0 * mm(A, z1) - z_nm
        return z_nm, z1, z2

    def rows(z_nm, c):
        return (z_nm.reshape(N, BC, c) + 0.0).reshape(N * BC, c)

    def nm(z_rows, c):
        return (z_rows.reshape(N, BC, c) + 0.0).reshape(N, BC * c)

    h0 = jnp.transpose(hid_ref[0].reshape(BC, N, U), (1, 0, 2))
    h0_rows = h0.reshape(N * BC, U)

    x_nm = jnp.transpose(inp_ref[...], (1, 0))
    x_taps = diffuse(x_nm)

    def gconv0(s_rows, kx, wh, bias, out_c):
        s0, s1, s2 = diffuse(nm(s_rows, U))
        acc = jnp.broadcast_to(bias[None, :], (N * BC, out_c))
        for m, s in enumerate((s0, s1, s2)):
            acc = acc + mm(rows(s, U), wh[m])
            acc = acc + rows(mm(x_taps[m], kx[m]), out_c)
        return acc

    g = jax.nn.sigmoid(gconv0(h0_rows, kxg0_ref[...], whg0_ref[...],
                              bg0_ref[...], 2 * U))
    r, u = g[:, :U], g[:, U:]
    c = jnp.tanh(gconv0(r * h0_rows, kxc0_ref[...], whc0_ref[...],
                        bc0_ref[...], U))
    h0n = u * h0_rows + (1.0 - u) * c

    h1 = jnp.transpose(hid_ref[1].reshape(BC, N, U), (1, 0, 2))
    h1_rows = h1.reshape(N * BC, U)
    x1_taps = diffuse(nm(h0n, U))

    def gconv1(s_rows, wx, wh, bias, out_c):
        s0, s1, s2 = diffuse(nm(s_rows, U))
        acc = jnp.broadcast_to(bias[None, :], (N * BC, out_c))
        for m, (s, xm) in enumerate(zip((s0, s1, s2), x1_taps)):
            acc = acc + mm(rows(s, U), wh[m]) + mm(rows(xm, U), wx[m])
        return acc

    g = jax.nn.sigmoid(gconv1(h1_rows, wxg1_ref[...], whg1_ref[...],
                              bg1_ref[...], 2 * U))
    r, u = g[:, :U], g[:, U:]
    c = jnp.tanh(gconv1(r * h1_rows, wxc1_ref[...], whc1_ref[...],
                        bc1_ref[...], U))
    h1n = u * h1_rows + (1.0 - u) * c

    hs_ref[0] = jnp.transpose(h0n.reshape(N, BC, U), (1, 0, 2)).reshape(BC, N * U)
    hs_ref[1] = jnp.transpose(h1n.reshape(N, BC, U), (1, 0, 2)).reshape(BC, N * U)
    proj = jnp.sum(h1n.reshape(N, BC, U) * wp_ref[...][None, :, :],
                   axis=-1) + bp_ref[0]
    out_ref[...] = jnp.transpose(proj, (1, 0))


def kernel(inputs, adj_mx, hidden_state, Wg0, bg0, Wc0, bc0,
           Wg1, bg1, Wc1, bc1, Wp, bp):
    wxg0, whg0 = _split_w(Wg0, 1)
    wxc0, whc0 = _split_w(Wc0, 1)
    kxg0 = _kron_x0(wxg0)
    kxc0 = _kron_x0(wxc0)
    wxg1, whg1 = _split_w(Wg1, U)
    wxc1, whc1 = _split_w(Wc1, U)

    full = lambda *shape: pl.BlockSpec(shape, lambda i: (0,) * len(shape))
    out, hs = pl.pallas_call(
        _dcgru_body,
        grid=(B // BC,),
        in_specs=[
            pl.BlockSpec((BC, N), lambda i: (i, 0)),
            full(N, N),
            pl.BlockSpec((2, BC, N * U), lambda i: (0, i, 0)),
            full(NTAP, BC, BC * 2 * U),
            full(NTAP, U, 2 * U),
            full(NTAP, BC, BC * U),
            full(NTAP, U, U),
            full(2 * U), full(U),
            full(NTAP, U, 2 * U),
            full(NTAP, U, 2 * U),
            full(NTAP, U, U),
            full(NTAP, U, U),
            full(2 * U), full(U),
            full(1, U), full(1),
        ],
        out_specs=(
            pl.BlockSpec((BC, N), lambda i: (i, 0)),
            pl.BlockSpec((2, BC, N * U), lambda i: (0, i, 0)),
        ),
        out_shape=(
            jax.ShapeDtypeStruct((B, N), F32),
            jax.ShapeDtypeStruct((2, B, N * U), F32),
        ),
    )(inputs, adj_mx, hidden_state,
      kxg0, whg0, kxc0, whc0, bg0, bc0,
      wxg1, whg1, wxc1, whc1, bg1, bc1,
      Wp.T, bp)
    return out, hs
